# baseline (speedup 1.0000x reference)
import functools

import numpy as np
import jax
import jax.numpy as jnp
from jax.experimental import pallas as pl
from jax.experimental.pallas import tpu as pltpu

_LANES = 128


_BIG = 1.0e30


def _bbox_pack_kernel(c_ref, sc_ref, sem_ref, cf_ref, out_ref,
                      *, score_thre):
    v = c_ref[...]
    lane = jax.lax.broadcasted_iota(jnp.int32, v.shape, 1) % 3
    m0 = lane == 0
    m1 = lane == 1
    m2 = lane == 2
    xmn = jnp.min(jnp.where(m0, v, _BIG), axis=1, keepdims=True)
    xmx = jnp.max(jnp.where(m0, v, -_BIG), axis=1, keepdims=True)
    ymn = jnp.min(jnp.where(m1, v, _BIG), axis=1, keepdims=True)
    ymx = jnp.max(jnp.where(m1, v, -_BIG), axis=1, keepdims=True)
    zmn = jnp.min(jnp.where(m2, v, _BIG), axis=1, keepdims=True)
    zmx = jnp.max(jnp.where(m2, v, -_BIG), axis=1, keepdims=True)

    cx = (xmn + xmx) * 0.5
    cy = (ymn + ymx) * 0.5
    cz = (zmn + zmx) * 0.5
    dx = xmx - xmn
    dy = ymx - ymn
    dz = zmx - zmn

    sig = jax.nn.sigmoid(sc_ref[...])
    msk = (sig > score_thre).astype(jnp.float32)
    sem = sem_ref[...]

    cm = cf_ref[...]
    out_ref[...] = (cx * cm[0:1, :] + cy * cm[1:2, :] + cz * cm[2:3, :]
                    + dx * cm[3:4, :] + dy * cm[4:5, :] + dz * cm[5:6, :]
                    + sig * cm[6:7, :] + msk * cm[7:8, :] + sem * cm[8:9, :])


def _build_coeff():
    xs = np.array([1, 1, -1, -1, 1, 1, -1, -1], np.float32)
    ys = np.array([1, 1, 1, 1, -1, -1, -1, -1], np.float32)
    zs = np.array([1, -1, -1, 1, 1, -1, -1, 1], np.float32)
    c = np.zeros((16, _LANES), np.float32)
    for j in range(8):
        c[0, 3 * j + 0] = 1.0
        c[3, 3 * j + 0] = xs[j] * 0.5
        c[1, 3 * j + 1] = 1.0
        c[5, 3 * j + 1] = ys[j] * 0.5
        c[2, 3 * j + 2] = 1.0
        c[4, 3 * j + 2] = zs[j] * 0.5
    for j in range(6):
        c[j, 24 + j] = 1.0
    c[6, 30] = 1.0
    c[7, 31] = 1.0
    c[8, 32] = 1.0
    return jnp.asarray(c)


def _run_bbox(coords, scores, sems, coeff, p_tile, score_thre):
    P, K3 = coords.shape
    grid = (P // p_tile,)
    coord_spec = pl.BlockSpec((p_tile, K3), lambda p: (p, 0))
    col_spec = pl.BlockSpec((p_tile, 1), lambda p: (p, 0))
    return pl.pallas_call(
        functools.partial(_bbox_pack_kernel, score_thre=score_thre),
        out_shape=jax.ShapeDtypeStruct((P, _LANES), jnp.float32),
        grid=grid,
        in_specs=[coord_spec, col_spec, col_spec,
                  pl.BlockSpec((16, _LANES), lambda p: (0, 0))],
        out_specs=pl.BlockSpec((p_tile, _LANES), lambda p: (p, 0)),
        compiler_params=pltpu.CompilerParams(
            dimension_semantics=("parallel",),
            vmem_limit_bytes=48 * 1024 * 1024),
    )(coords, scores, sems, coeff)


def _gather_kernel(src_ref, feat_ref, ext_ref, fout_ref, eout_ref,
                   *, r_chunk, n_src):
    base = pl.program_id(0) * r_chunk
    for i in range(r_chunk):
        idx = src_ref[base + i]
        safe = jnp.minimum(idx, n_src - 1)
        vf = (idx < n_src).astype(jnp.float32)
        fout_ref[i] = feat_ref[safe] * vf
        eout_ref[i] = ext_ref[safe] * vf


def _run_gather(src_rows, feats3, ext3, r_chunk):
    n_src, _, c = feats3.shape
    rpad = src_rows.shape[0]
    grid_spec = pltpu.PrefetchScalarGridSpec(
        num_scalar_prefetch=1,
        grid=(rpad // r_chunk,),
        in_specs=[
            pl.BlockSpec((n_src, 1, c), lambda r, src: (0, 0, 0)),
            pl.BlockSpec((n_src, 1, _LANES), lambda r, src: (0, 0, 0)),
        ],
        out_specs=[
            pl.BlockSpec((r_chunk, 1, c), lambda r, src: (r, 0, 0)),
            pl.BlockSpec((r_chunk, 1, _LANES), lambda r, src: (r, 0, 0)),
        ],
    )
    return pl.pallas_call(
        functools.partial(_gather_kernel, r_chunk=r_chunk, n_src=n_src),
        out_shape=[jax.ShapeDtypeStruct((rpad, 1, c), jnp.float32),
                   jax.ShapeDtypeStruct((rpad, 1, _LANES), jnp.float32)],
        grid_spec=grid_spec,
        compiler_params=pltpu.CompilerParams(
            dimension_semantics=("parallel",),
            vmem_limit_bytes=48 * 1024 * 1024),
    )(src_rows, feats3, ext3)


def _capnet(locs_float, proposal_feats, proposals_idx, proposals_offset,
            proposal_scores, semantic_preds, batch_offsets,
            batch_size, num_proposal, score_thre):
    P = int(proposals_offset.shape[0]) - 1
    M = int(proposals_idx.shape[0])
    C = int(proposal_feats.shape[1])
    K = M // P

    p_tile = 256
    while P % p_tile:
        p_tile //= 2

    pt_ids = proposals_idx[:, 1]
    coords = locs_float[pt_ids].reshape(P, 3 * K)

    scores = proposal_scores.reshape(P, 1).astype(jnp.float32)
    first_pts = pt_ids.reshape(P, K)[:, 0]
    sems = semantic_preds[first_pts].astype(jnp.float32).reshape(P, 1)

    packed = _run_bbox(coords, scores, sems, _build_coeff(),
                       p_tile, score_thre)

    batch_id = (jnp.searchsorted(batch_offsets, first_pts, side="right") - 1
                ).astype(jnp.int32)
    onehot = (batch_id[:, None] ==
              jnp.arange(batch_size, dtype=jnp.int32)[None, :]).astype(jnp.int32)
    cum = jnp.cumsum(onehot, axis=0)
    slot = jnp.take_along_axis(cum, batch_id[:, None], axis=1)[:, 0] - 1
    valid_slot = slot < num_proposal

    R = batch_size * num_proposal
    rows = batch_id * num_proposal + slot
    scatter_rows = jnp.where(valid_slot, rows, R)
    src_rows = jnp.full((R,), P, jnp.int32).at[scatter_rows].set(
        jnp.arange(P, dtype=jnp.int32), mode="drop")

    r_chunk = 64
    while R % r_chunk:
        r_chunk //= 2

    feats3 = proposal_feats.astype(jnp.float32).reshape(P, 1, C)
    ext3 = packed.reshape(P, 1, _LANES)
    fout, eout = _run_gather(src_rows, feats3, ext3, r_chunk)

    feat = fout.reshape(batch_size, num_proposal, C)
    ext = eout.reshape(batch_size, num_proposal, _LANES)

    out = {}
    out["bbox_feature"] = feat
    out["bbox_corner"] = ext[..., :24].reshape(batch_size, num_proposal, 8, 3)
    out["bbox_parameters"] = ext[..., 24:30]
    out["bbox_scores"] = ext[..., 30]
    out["bbox_mask"] = ext[..., 31]
    out["bbox_sems"] = ext[..., 32]
    out["sem_cls"] = out["bbox_sems"]
    return out


def kernel(locs_float, proposal_feats, proposals_idx, proposals_offset,
           proposal_scores, semantic_preds, batch_offsets):
    return _capnet(locs_float, proposal_feats, proposals_idx, proposals_offset,
                   proposal_scores, semantic_preds, batch_offsets,
                   batch_size=8, num_proposal=256, score_thre=0.09)


# speedup vs baseline: 2.0648x; 2.0648x over previous
import functools

import numpy as np
import jax
import jax.numpy as jnp
from jax.experimental import pallas as pl
from jax.experimental.pallas import tpu as pltpu

_LANES = 128


def _bbox_pack_kernel(x_ref, y_ref, z_ref, sc_ref, sem_ref, cf_ref, out_ref,
                      *, score_thre):
    xmn = jnp.min(x_ref[...], axis=1, keepdims=True)
    xmx = jnp.max(x_ref[...], axis=1, keepdims=True)
    ymn = jnp.min(y_ref[...], axis=1, keepdims=True)
    ymx = jnp.max(y_ref[...], axis=1, keepdims=True)
    zmn = jnp.min(z_ref[...], axis=1, keepdims=True)
    zmx = jnp.max(z_ref[...], axis=1, keepdims=True)

    cx = (xmn + xmx) * 0.5
    cy = (ymn + ymx) * 0.5
    cz = (zmn + zmx) * 0.5
    dx = xmx - xmn
    dy = ymx - ymn
    dz = zmx - zmn

    sig = jax.nn.sigmoid(sc_ref[...])
    msk = (sig > score_thre).astype(jnp.float32)
    sem = sem_ref[...]

    cm = cf_ref[...]
    out_ref[...] = (cx * cm[0:1, :] + cy * cm[1:2, :] + cz * cm[2:3, :]
                    + dx * cm[3:4, :] + dy * cm[4:5, :] + dz * cm[5:6, :]
                    + sig * cm[6:7, :] + msk * cm[7:8, :] + sem * cm[8:9, :])


def _build_coeff():
    xs = np.array([1, 1, -1, -1, 1, 1, -1, -1], np.float32)
    ys = np.array([1, 1, 1, 1, -1, -1, -1, -1], np.float32)
    zs = np.array([1, -1, -1, 1, 1, -1, -1, 1], np.float32)
    c = np.zeros((16, _LANES), np.float32)
    for j in range(8):
        c[0, 3 * j + 0] = 1.0
        c[3, 3 * j + 0] = xs[j] * 0.5
        c[1, 3 * j + 1] = 1.0
        c[5, 3 * j + 1] = ys[j] * 0.5
        c[2, 3 * j + 2] = 1.0
        c[4, 3 * j + 2] = zs[j] * 0.5
    for j in range(6):
        c[j, 24 + j] = 1.0
    c[6, 30] = 1.0
    c[7, 31] = 1.0
    c[8, 32] = 1.0
    return jnp.asarray(c)


def _run_bbox(xs, ys, zs, scores, sems, coeff, p_tile, score_thre):
    P, K = xs.shape
    grid = (P // p_tile,)
    coord_spec = pl.BlockSpec((p_tile, K), lambda p: (p, 0))
    col_spec = pl.BlockSpec((p_tile, 1), lambda p: (p, 0))
    return pl.pallas_call(
        functools.partial(_bbox_pack_kernel, score_thre=score_thre),
        out_shape=jax.ShapeDtypeStruct((P, _LANES), jnp.float32),
        grid=grid,
        in_specs=[coord_spec, coord_spec, coord_spec, col_spec, col_spec,
                  pl.BlockSpec((16, _LANES), lambda p: (0, 0))],
        out_specs=pl.BlockSpec((p_tile, _LANES), lambda p: (p, 0)),
        compiler_params=pltpu.CompilerParams(
            dimension_semantics=("parallel",),
            vmem_limit_bytes=48 * 1024 * 1024),
    )(xs, ys, zs, scores, sems, coeff)


def _gather_kernel(src_ref, feat_ref, ext_ref, fout_ref, eout_ref,
                   *, r_chunk, n_src):
    base = pl.program_id(0) * r_chunk
    for i in range(r_chunk):
        idx = src_ref[base + i]
        safe = jnp.minimum(idx, n_src - 1)
        vf = (idx < n_src).astype(jnp.float32)
        fout_ref[i] = feat_ref[safe] * vf
        eout_ref[i] = ext_ref[safe] * vf


def _run_gather(src_rows, feats3, ext3, r_chunk):
    n_src, _, c = feats3.shape
    rpad = src_rows.shape[0]
    grid_spec = pltpu.PrefetchScalarGridSpec(
        num_scalar_prefetch=1,
        grid=(rpad // r_chunk,),
        in_specs=[
            pl.BlockSpec((n_src, 1, c), lambda r, src: (0, 0, 0)),
            pl.BlockSpec((n_src, 1, _LANES), lambda r, src: (0, 0, 0)),
        ],
        out_specs=[
            pl.BlockSpec((r_chunk, 1, c), lambda r, src: (r, 0, 0)),
            pl.BlockSpec((r_chunk, 1, _LANES), lambda r, src: (r, 0, 0)),
        ],
    )
    return pl.pallas_call(
        functools.partial(_gather_kernel, r_chunk=r_chunk, n_src=n_src),
        out_shape=[jax.ShapeDtypeStruct((rpad, 1, c), jnp.float32),
                   jax.ShapeDtypeStruct((rpad, 1, _LANES), jnp.float32)],
        grid_spec=grid_spec,
        compiler_params=pltpu.CompilerParams(
            dimension_semantics=("parallel",),
            vmem_limit_bytes=48 * 1024 * 1024),
    )(src_rows, feats3, ext3)


def _capnet(locs_float, proposal_feats, proposals_idx, proposals_offset,
            proposal_scores, semantic_preds, batch_offsets,
            batch_size, num_proposal, score_thre):
    P = int(proposals_offset.shape[0]) - 1
    M = int(proposals_idx.shape[0])
    C = int(proposal_feats.shape[1])
    K = M // P

    p_tile = 256
    while P % p_tile:
        p_tile //= 2

    pt_ids = proposals_idx[:, 1]
    xs = locs_float[pt_ids, 0].reshape(P, K)
    ys = locs_float[pt_ids, 1].reshape(P, K)
    zs = locs_float[pt_ids, 2].reshape(P, K)

    scores = proposal_scores.reshape(P, 1).astype(jnp.float32)
    first_pts = pt_ids.reshape(P, K)[:, 0]
    sems = semantic_preds[first_pts].astype(jnp.float32).reshape(P, 1)

    packed = _run_bbox(xs, ys, zs, scores, sems, _build_coeff(),
                       p_tile, score_thre)

    batch_id = (jnp.searchsorted(batch_offsets, first_pts, side="right") - 1
                ).astype(jnp.int32)
    onehot = (batch_id[:, None] ==
              jnp.arange(batch_size, dtype=jnp.int32)[None, :]).astype(jnp.int32)
    cum = jnp.cumsum(onehot, axis=0)
    slot = jnp.take_along_axis(cum, batch_id[:, None], axis=1)[:, 0] - 1
    valid_slot = slot < num_proposal

    R = batch_size * num_proposal
    rows = batch_id * num_proposal + slot
    scatter_rows = jnp.where(valid_slot, rows, R)
    src_rows = jnp.full((R,), P, jnp.int32).at[scatter_rows].set(
        jnp.arange(P, dtype=jnp.int32), mode="drop")

    r_chunk = 64
    while R % r_chunk:
        r_chunk //= 2

    feats3 = proposal_feats.astype(jnp.float32).reshape(P, 1, C)
    ext3 = packed.reshape(P, 1, _LANES)
    fout, eout = _run_gather(src_rows, feats3, ext3, r_chunk)

    feat = fout.reshape(batch_size, num_proposal, C)
    ext = eout.reshape(batch_size, num_proposal, _LANES)

    out = {}
    out["bbox_feature"] = feat
    out["bbox_corner"] = ext[..., :24].reshape(batch_size, num_proposal, 8, 3)
    out["bbox_parameters"] = ext[..., 24:30]
    out["bbox_scores"] = ext[..., 30]
    out["bbox_mask"] = ext[..., 31]
    out["bbox_sems"] = ext[..., 32]
    out["sem_cls"] = out["bbox_sems"]
    return out


def kernel(locs_float, proposal_feats, proposals_idx, proposals_offset,
           proposal_scores, semantic_preds, batch_offsets):
    return _capnet(locs_float, proposal_feats, proposals_idx, proposals_offset,
                   proposal_scores, semantic_preds, batch_offsets,
                   batch_size=8, num_proposal=256, score_thre=0.09)


# speedup vs baseline: 2.4657x; 1.1942x over previous
import functools

import numpy as np
import jax
import jax.numpy as jnp
from jax.experimental import pallas as pl
from jax.experimental.pallas import tpu as pltpu

_LANES = 128


def _fused_kernel(src_ref, x_ref, y_ref, z_ref, sc_ref, sem_ref, vm_ref,
                  cf_ref, feat_ref, fout_ref, eout_ref,
                  *, r_chunk, n_src, score_thre):
    xmn = jnp.min(x_ref[...], axis=1, keepdims=True)
    xmx = jnp.max(x_ref[...], axis=1, keepdims=True)
    ymn = jnp.min(y_ref[...], axis=1, keepdims=True)
    ymx = jnp.max(y_ref[...], axis=1, keepdims=True)
    zmn = jnp.min(z_ref[...], axis=1, keepdims=True)
    zmx = jnp.max(z_ref[...], axis=1, keepdims=True)

    cx = (xmn + xmx) * 0.5
    cy = (ymn + ymx) * 0.5
    cz = (zmn + zmx) * 0.5
    dx = xmx - xmn
    dy = ymx - ymn
    dz = zmx - zmn

    sig = jax.nn.sigmoid(sc_ref[...])
    msk = (sig > score_thre).astype(jnp.float32)
    sem = sem_ref[...]

    cm = cf_ref[...]
    packed = (cx * cm[0:1, :] + cy * cm[1:2, :] + cz * cm[2:3, :]
              + dx * cm[3:4, :] + dy * cm[4:5, :] + dz * cm[5:6, :]
              + sig * cm[6:7, :] + msk * cm[7:8, :] + sem * cm[8:9, :])
    eout_ref[...] = packed * vm_ref[...]

    base = pl.program_id(0) * r_chunk
    for i in range(r_chunk):
        idx = src_ref[base + i]
        safe = jnp.minimum(idx, n_src - 1)
        vf = (idx < n_src).astype(jnp.float32)
        fout_ref[i] = feat_ref[safe] * vf


def _build_coeff():
    xs = np.array([1, 1, -1, -1, 1, 1, -1, -1], np.float32)
    ys = np.array([1, 1, 1, 1, -1, -1, -1, -1], np.float32)
    zs = np.array([1, -1, -1, 1, 1, -1, -1, 1], np.float32)
    c = np.zeros((16, _LANES), np.float32)
    for j in range(8):
        c[0, 3 * j + 0] = 1.0
        c[3, 3 * j + 0] = xs[j] * 0.5
        c[1, 3 * j + 1] = 1.0
        c[5, 3 * j + 1] = ys[j] * 0.5
        c[2, 3 * j + 2] = 1.0
        c[4, 3 * j + 2] = zs[j] * 0.5
    for j in range(6):
        c[j, 24 + j] = 1.0
    c[6, 30] = 1.0
    c[7, 31] = 1.0
    c[8, 32] = 1.0
    return jnp.asarray(c)


def _run_fused(src_rows, xs, ys, zs, scores, sems, vmask, coeff, feats3,
               r_chunk, score_thre):
    n_src, _, c = feats3.shape
    rpad, k = xs.shape
    coord_spec = pl.BlockSpec((r_chunk, k), lambda r, src: (r, 0))
    col_spec = pl.BlockSpec((r_chunk, 1), lambda r, src: (r, 0))
    grid_spec = pltpu.PrefetchScalarGridSpec(
        num_scalar_prefetch=1,
        grid=(rpad // r_chunk,),
        in_specs=[
            coord_spec, coord_spec, coord_spec, col_spec, col_spec, col_spec,
            pl.BlockSpec((16, _LANES), lambda r, src: (0, 0)),
            pl.BlockSpec((n_src, 1, c), lambda r, src: (0, 0, 0)),
        ],
        out_specs=[
            pl.BlockSpec((r_chunk, 1, c), lambda r, src: (r, 0, 0)),
            pl.BlockSpec((r_chunk, _LANES), lambda r, src: (r, 0)),
        ],
    )
    return pl.pallas_call(
        functools.partial(_fused_kernel, r_chunk=r_chunk, n_src=n_src,
                          score_thre=score_thre),
        out_shape=[jax.ShapeDtypeStruct((rpad, 1, c), jnp.float32),
                   jax.ShapeDtypeStruct((rpad, _LANES), jnp.float32)],
        grid_spec=grid_spec,
        compiler_params=pltpu.CompilerParams(
            dimension_semantics=("parallel",),
            vmem_limit_bytes=48 * 1024 * 1024),
    )(src_rows, xs, ys, zs, scores, sems, vmask, coeff, feats3)


def _capnet(locs_float, proposal_feats, proposals_idx, proposals_offset,
            proposal_scores, semantic_preds, batch_offsets,
            batch_size, num_proposal, score_thre):
    P = int(proposals_offset.shape[0]) - 1
    M = int(proposals_idx.shape[0])
    C = int(proposal_feats.shape[1])
    K = M // P

    pt_ids = proposals_idx[:, 1]
    pt_grid = pt_ids.reshape(P, K)
    first_pts = pt_grid[:, 0]
    batch_id = (jnp.searchsorted(batch_offsets, first_pts, side="right") - 1
                ).astype(jnp.int32)
    onehot = (batch_id[:, None] ==
              jnp.arange(batch_size, dtype=jnp.int32)[None, :]).astype(jnp.int32)
    cum = jnp.cumsum(onehot, axis=0)
    slot = jnp.take_along_axis(cum, batch_id[:, None], axis=1)[:, 0] - 1
    valid_slot = slot < num_proposal

    R = batch_size * num_proposal
    rows = batch_id * num_proposal + slot
    scatter_rows = jnp.where(valid_slot, rows, R)
    src_rows = jnp.full((R,), P, jnp.int32).at[scatter_rows].set(
        jnp.arange(P, dtype=jnp.int32), mode="drop")

    src_safe = jnp.minimum(src_rows, P - 1)
    mem_ids = pt_grid[src_safe]
    xs = locs_float[mem_ids, 0]
    ys = locs_float[mem_ids, 1]
    zs = locs_float[mem_ids, 2]

    scores = proposal_scores.reshape(P)[src_safe].reshape(R, 1)
    sems = semantic_preds[mem_ids[:, 0]].astype(jnp.float32).reshape(R, 1)
    vmask = (src_rows < P).astype(jnp.float32).reshape(R, 1)

    r_chunk = 64
    while R % r_chunk:
        r_chunk //= 2

    feats3 = proposal_feats.astype(jnp.float32).reshape(P, 1, C)
    fout, eout = _run_fused(src_rows, xs, ys, zs, scores, sems, vmask,
                            _build_coeff(), feats3, r_chunk, score_thre)

    feat = fout.reshape(batch_size, num_proposal, C)
    ext = eout.reshape(batch_size, num_proposal, _LANES)

    out = {}
    out["bbox_feature"] = feat
    out["bbox_corner"] = ext[..., :24].reshape(batch_size, num_proposal, 8, 3)
    out["bbox_parameters"] = ext[..., 24:30]
    out["bbox_scores"] = ext[..., 30]
    out["bbox_mask"] = ext[..., 31]
    out["bbox_sems"] = ext[..., 32]
    out["sem_cls"] = out["bbox_sems"]
    return out


def kernel(locs_float, proposal_feats, proposals_idx, proposals_offset,
           proposal_scores, semantic_preds, batch_offsets):
    return _capnet(locs_float, proposal_feats, proposals_idx, proposals_offset,
                   proposal_scores, semantic_preds, batch_offsets,
                   batch_size=8, num_proposal=256, score_thre=0.09)
